# baseline (speedup 1.0000x reference)
import functools

import jax
import jax.numpy as jnp
import numpy as np
from jax import lax
from jax.experimental import pallas as pl
from jax.experimental.pallas import tpu as pltpu
from jax.experimental.pallas import tpu_sc as plsc

NC = 2
NS = 16
NW = NC * NS
CHUNK = 128
BLOCK = 8
SPLIT0 = 0.54


def _linear_tc(x, W, b):
    N, D_in = x.shape
    D_out = W.shape[0]
    BLK = 1000
    grid = (N // BLK,)

    def body(x_ref, w_ref, b_ref, h_ref):
        acc = lax.dot_general(
            x_ref[...], w_ref[...],
            (((1,), (1,)), ((), ())),
            preferred_element_type=jnp.float32,
        )
        h_ref[...] = acc + b_ref[...][None, :]

    return pl.pallas_call(
        body,
        grid=grid,
        in_specs=[
            pl.BlockSpec((BLK, D_in), lambda i: (i, 0)),
            pl.BlockSpec((D_out, D_in), lambda i: (0, 0)),
            pl.BlockSpec((D_out,), lambda i: (0,)),
        ],
        out_specs=pl.BlockSpec((BLK, D_out), lambda i: (i, 0)),
        out_shape=jax.ShapeDtypeStruct((N, D_out), jnp.float32),
    )(x, W, b)


def _combine_tc(part):
    _, N, D = part.shape
    BLK = 1000
    grid = (N // BLK,)

    def body(p_ref, o_ref):
        o_ref[...] = p_ref[0] + p_ref[1]

    return pl.pallas_call(
        body,
        grid=grid,
        in_specs=[pl.BlockSpec((2, BLK, D), lambda i: (0, i, 0))],
        out_specs=pl.BlockSpec((BLK, D), lambda i: (i, 0)),
        out_shape=jax.ShapeDtypeStruct((N, D), jnp.float32),
    )(part)


def _aggregate_sc(h, srcr, dstr, wr, n0, n1, N, D):
    rows_per_tile = (N // NS) // 8 * 8
    rem_rows = N - rows_per_tile * NS

    mesh = plsc.VectorSubcoreMesh(core_axis_name="c", subcore_axis_name="s",
                                  num_cores=NC, num_subcores=NS)

    @functools.partial(
        pl.kernel,
        out_type=jax.ShapeDtypeStruct((NC, N, D), jnp.float32),
        mesh=mesh,
        scratch_types=[
            pltpu.VMEM_SHARED((N, D), jnp.float32),
            pltpu.VMEM((2, BLOCK, CHUNK), jnp.int32),
            pltpu.VMEM((2, BLOCK, CHUNK), jnp.int32),
            pltpu.VMEM((2, BLOCK, CHUNK), jnp.float32),
            pltpu.VMEM((2, CHUNK, D), jnp.float32),
            pltpu.SemaphoreType.DMA,
            pltpu.SemaphoreType.DMA,
            pltpu.SemaphoreType.DMA,
            pltpu.SemaphoreType.DMA,
            pltpu.SemaphoreType.DMA,
        ],
    )
    def k(h_hbm, src_hbm, dst_hbm, w_hbm, part_hbm,
          acc, src_v, dst_v, w_v, rows_v,
          sem_i, sem_g0, sem_g1, sem_s0, sem_s1):
        cid = lax.axis_index("c")
        sid = lax.axis_index("s")
        nb0 = n0 // BLOCK
        nb1 = n1 // BLOCK
        n_t = jnp.where(cid == 0, n0, n1)
        bstart = jnp.where(cid == 0, sid * nb0, NS * nb0 + sid * nb1)
        sem_g = (sem_g0, sem_g1)
        sem_s = (sem_s0, sem_s1)

        def issue_idxblk(k_):
            kb = k_ % 2
            pltpu.async_copy(src_hbm.at[bstart + k_], src_v.at[kb], sem_i)
            pltpu.async_copy(dst_hbm.at[bstart + k_], dst_v.at[kb], sem_i)
            pltpu.async_copy(w_hbm.at[bstart + k_], w_v.at[kb], sem_i)

        def wait_idxblk():
            pltpu.make_async_copy(src_hbm.at[0], src_v.at[0], sem_i).wait()
            pltpu.make_async_copy(dst_hbm.at[0], dst_v.at[0], sem_i).wait()
            pltpu.make_async_copy(w_hbm.at[0], w_v.at[0], sem_i).wait()

        def issue_gather(kb, j, b):
            pltpu.async_copy(h_hbm.at[src_v.at[kb, j]], rows_v.at[b],
                             sem_g[b])

        def wait_gather(b):
            pltpu.make_async_copy(h_hbm.at[pl.ds(0, CHUNK)], rows_v.at[b],
                                  sem_g[b]).wait()

        def issue_scatter(kb, j, b):
            pltpu.async_copy(rows_v.at[b], acc.at[dst_v.at[kb, j]],
                             sem_s[b], add=True)

        def wait_scatter(b):
            pltpu.make_async_copy(h_hbm.at[pl.ds(0, CHUNK)], rows_v.at[b],
                                  sem_s[b]).wait()

        @pl.when(n_t > 0)
        def _():
            issue_idxblk(0)

        def zfill(i, _):
            r = i // (D // 16)
            c = (i % (D // 16)) * 16
            rows_v[0, r, pl.ds(c, 16)] = jnp.zeros((16,), jnp.float32)
            return 0
        lax.fori_loop(0, CHUNK * (D // 16), zfill, 0)

        base = sid * rows_per_tile
        full = rows_per_tile // CHUNK
        rem = rows_per_tile - full * CHUNK
        zcopies = []
        for q in range(full):
            zcopies.append(pltpu.async_copy(
                rows_v.at[0], acc.at[pl.ds(base + q * CHUNK, CHUNK)],
                sem_g0))
        if rem:
            zcopies.append(pltpu.async_copy(
                rows_v.at[0, pl.ds(0, rem)],
                acc.at[pl.ds(base + full * CHUNK, rem)], sem_g0))
        if rem_rows:
            @pl.when(sid == NS - 1)
            def _():
                pltpu.sync_copy(rows_v.at[0, pl.ds(0, rem_rows)],
                                acc.at[pl.ds(NS * rows_per_tile, rem_rows)])
        for cp in zcopies:
            cp.wait()

        plsc.subcore_barrier()

        @pl.when(n_t > 0)
        def _():
            wait_idxblk()
            issue_gather(0, 0, 0)

        def scale_rows(kb, j, b):
            def group_body(g, _):
                wv16 = w_v[kb, j, pl.ds(g * 16, 16)]
                for t in range(16):
                    e = g * 16 + t
                    wgt = wv16[t]
                    for u in range(D // 16):
                        sl = pl.ds(u * 16, 16)
                        rows_v[b, e, sl] = rows_v[b, e, sl] * wgt
                return 0
            lax.fori_loop(0, CHUNK // 16, group_body, 0)

        def outer_body(k_, _):
            kb = k_ % 2
            for j in range(BLOCK):
                c = BLOCK * k_ + j
                b = j % 2

                @pl.when(c >= 1)
                def _():
                    wait_scatter(1 - b)

                if j == 0:
                    @pl.when(BLOCK * (k_ + 1) < n_t)
                    def _():
                        issue_idxblk(k_ + 1)

                if j == BLOCK - 1:
                    @pl.when(c + 1 < n_t)
                    def _():
                        wait_idxblk()

                @pl.when(c + 1 < n_t)
                def _():
                    if j == BLOCK - 1:
                        issue_gather(1 - kb, 0, 1 - b)
                    else:
                        issue_gather(kb, j + 1, 1 - b)

                wait_gather(b)
                scale_rows(kb, j, b)
                issue_scatter(kb, j, b)
            return 0
        lax.fori_loop(0, n_t // BLOCK, outer_body, 0)

        @pl.when(n_t > 0)
        def _():
            wait_scatter((BLOCK - 1) % 2)

        plsc.subcore_barrier()

        pltpu.sync_copy(acc.at[pl.ds(base, rows_per_tile)],
                        part_hbm.at[cid, pl.ds(base, rows_per_tile)])
        if rem_rows:
            @pl.when(sid == NS - 1)
            def _():
                tail = NS * rows_per_tile
                pltpu.sync_copy(acc.at[pl.ds(tail, rem_rows)],
                                part_hbm.at[cid, pl.ds(tail, rem_rows)])

    return k(h, srcr, dstr, wr)


def kernel(x, edge_index, edge_weight, W, b):
    N, _ = x.shape
    D = W.shape[0]
    E = edge_weight.shape[0]

    h = _linear_tc(x, W, b)

    t_chunks = -(-E // CHUNK)

    def _blk_pt(chunks):
        pt = -(-chunks // NS)
        return -(-pt // BLOCK) * BLOCK

    n0 = _blk_pt(int(round(t_chunks * SPLIT0)))
    n1 = _blk_pt(max(t_chunks - NS * n0, 0))
    e_pad = NS * (n0 + n1) * CHUNK
    pad_n = e_pad - E
    pad_idx = jnp.asarray((np.arange(pad_n, dtype=np.int32) * 13) % N)
    dst = jnp.concatenate([edge_index[0], pad_idx])
    src = jnp.concatenate([edge_index[1], pad_idx])
    w = jnp.pad(edge_weight, (0, pad_n))
    srcr = src.reshape(-1, BLOCK, CHUNK)
    dstr = dst.reshape(-1, BLOCK, CHUNK)
    wr = w.reshape(-1, BLOCK, CHUNK)

    part = _aggregate_sc(h, srcr, dstr, wr, n0, n1, N, D)
    return _combine_tc(part)


# speedup vs baseline: 1.0419x; 1.0419x over previous
import functools

import jax
import jax.numpy as jnp
import numpy as np
from jax import lax
from jax.experimental import pallas as pl
from jax.experimental.pallas import tpu as pltpu
from jax.experimental.pallas import tpu_sc as plsc

NC = 2
NS = 16
NW = NC * NS
CHUNK = 112
BLOCK = 6
RING = 3
SPLIT0 = 0.54


def _linear_tc(x, W, b):
    N, D_in = x.shape
    D_out = W.shape[0]
    BLK = 1000
    grid = (N // BLK,)

    def body(x_ref, w_ref, b_ref, h_ref):
        acc = lax.dot_general(
            x_ref[...], w_ref[...],
            (((1,), (1,)), ((), ())),
            preferred_element_type=jnp.float32,
        )
        h_ref[...] = acc + b_ref[...][None, :]

    return pl.pallas_call(
        body,
        grid=grid,
        in_specs=[
            pl.BlockSpec((BLK, D_in), lambda i: (i, 0)),
            pl.BlockSpec((D_out, D_in), lambda i: (0, 0)),
            pl.BlockSpec((D_out,), lambda i: (0,)),
        ],
        out_specs=pl.BlockSpec((BLK, D_out), lambda i: (i, 0)),
        out_shape=jax.ShapeDtypeStruct((N, D_out), jnp.float32),
    )(x, W, b)


def _combine_tc(part):
    _, N, D = part.shape
    BLK = 1000
    grid = (N // BLK,)

    def body(p_ref, o_ref):
        o_ref[...] = p_ref[0] + p_ref[1]

    return pl.pallas_call(
        body,
        grid=grid,
        in_specs=[pl.BlockSpec((2, BLK, D), lambda i: (0, i, 0))],
        out_specs=pl.BlockSpec((BLK, D), lambda i: (i, 0)),
        out_shape=jax.ShapeDtypeStruct((N, D), jnp.float32),
    )(part)


def _aggregate_sc(h, srcr, dstr, wr, n0, n1, N, D):
    rows_per_tile = (N // NS) // 8 * 8
    rem_rows = N - rows_per_tile * NS

    mesh = plsc.VectorSubcoreMesh(core_axis_name="c", subcore_axis_name="s",
                                  num_cores=NC, num_subcores=NS)

    @functools.partial(
        pl.kernel,
        out_type=jax.ShapeDtypeStruct((NC, N, D), jnp.float32),
        mesh=mesh,
        scratch_types=[
            pltpu.VMEM_SHARED((N, D), jnp.float32),
            pltpu.VMEM((2, BLOCK, CHUNK), jnp.int32),
            pltpu.VMEM((2, BLOCK, CHUNK), jnp.int32),
            pltpu.VMEM((2, BLOCK, CHUNK), jnp.float32),
            pltpu.VMEM((RING, CHUNK, D), jnp.float32),
            pltpu.SemaphoreType.DMA,
            pltpu.SemaphoreType.DMA,
            pltpu.SemaphoreType.DMA,
            pltpu.SemaphoreType.DMA,
            pltpu.SemaphoreType.DMA,
            pltpu.SemaphoreType.DMA,
            pltpu.SemaphoreType.DMA,
        ],
    )
    def k(h_hbm, src_hbm, dst_hbm, w_hbm, part_hbm,
          acc, src_v, dst_v, w_v, rows_v,
          sem_i, sem_g0, sem_g1, sem_g2, sem_s0, sem_s1, sem_s2):
        cid = lax.axis_index("c")
        sid = lax.axis_index("s")
        nb0 = n0 // BLOCK
        nb1 = n1 // BLOCK
        n_t = jnp.where(cid == 0, n0, n1)
        bstart = jnp.where(cid == 0, sid * nb0, NS * nb0 + sid * nb1)
        sem_g = (sem_g0, sem_g1, sem_g2)
        sem_s = (sem_s0, sem_s1, sem_s2)

        def issue_idxblk(k_):
            kb = k_ % 2
            pltpu.async_copy(src_hbm.at[bstart + k_], src_v.at[kb], sem_i)
            pltpu.async_copy(dst_hbm.at[bstart + k_], dst_v.at[kb], sem_i)
            pltpu.async_copy(w_hbm.at[bstart + k_], w_v.at[kb], sem_i)

        def wait_idxblk():
            pltpu.make_async_copy(src_hbm.at[0], src_v.at[0], sem_i).wait()
            pltpu.make_async_copy(dst_hbm.at[0], dst_v.at[0], sem_i).wait()
            pltpu.make_async_copy(w_hbm.at[0], w_v.at[0], sem_i).wait()

        def issue_gather(kb, j, b):
            pltpu.async_copy(h_hbm.at[src_v.at[kb, j]], rows_v.at[b],
                             sem_g[b])

        def wait_gather(b):
            pltpu.make_async_copy(h_hbm.at[pl.ds(0, CHUNK)], rows_v.at[b],
                                  sem_g[b]).wait()

        def issue_scatter(kb, j, b):
            pltpu.async_copy(rows_v.at[b], acc.at[dst_v.at[kb, j]],
                             sem_s[b], add=True)

        def wait_scatter(b):
            pltpu.make_async_copy(h_hbm.at[pl.ds(0, CHUNK)], rows_v.at[b],
                                  sem_s[b]).wait()

        @pl.when(n_t > 0)
        def _():
            issue_idxblk(0)

        def zfill(i, _):
            r = i // (D // 16)
            c = (i % (D // 16)) * 16
            rows_v[0, r, pl.ds(c, 16)] = jnp.zeros((16,), jnp.float32)
            return 0
        lax.fori_loop(0, CHUNK * (D // 16), zfill, 0)

        base = sid * rows_per_tile
        full = rows_per_tile // CHUNK
        rem = rows_per_tile - full * CHUNK
        zcopies = []
        for q in range(full):
            zcopies.append(pltpu.async_copy(
                rows_v.at[0], acc.at[pl.ds(base + q * CHUNK, CHUNK)],
                sem_g0))
        if rem:
            zcopies.append(pltpu.async_copy(
                rows_v.at[0, pl.ds(0, rem)],
                acc.at[pl.ds(base + full * CHUNK, rem)], sem_g0))
        if rem_rows:
            @pl.when(sid == NS - 1)
            def _():
                pltpu.sync_copy(rows_v.at[0, pl.ds(0, rem_rows)],
                                acc.at[pl.ds(NS * rows_per_tile, rem_rows)])
        for cp in zcopies:
            cp.wait()

        plsc.subcore_barrier()

        @pl.when(n_t > 0)
        def _():
            wait_idxblk()
            issue_gather(0, 0, 0)

        def scale_rows(kb, j, b):
            def group_body(g, _):
                wv16 = w_v[kb, j, pl.ds(g * 16, 16)]
                for t in range(16):
                    e = g * 16 + t
                    wgt = wv16[t]
                    for u in range(D // 16):
                        sl = pl.ds(u * 16, 16)
                        rows_v[b, e, sl] = rows_v[b, e, sl] * wgt
                return 0
            lax.fori_loop(0, CHUNK // 16, group_body, 0)

        def outer_body(k_, _):
            kb = k_ % 2
            for j in range(BLOCK):
                c = BLOCK * k_ + j
                b = j % RING
                nxt = (j + 1) % RING

                @pl.when(c + 1 >= RING)
                def _():
                    wait_scatter(nxt)

                if j == 0:
                    @pl.when(BLOCK * (k_ + 1) < n_t)
                    def _():
                        issue_idxblk(k_ + 1)

                if j == BLOCK - 1:
                    @pl.when(c + 1 < n_t)
                    def _():
                        wait_idxblk()

                @pl.when(c + 1 < n_t)
                def _():
                    if j == BLOCK - 1:
                        issue_gather(1 - kb, 0, nxt)
                    else:
                        issue_gather(kb, j + 1, nxt)

                wait_gather(b)
                scale_rows(kb, j, b)
                issue_scatter(kb, j, b)
            return 0
        lax.fori_loop(0, n_t // BLOCK, outer_body, 0)

        @pl.when(n_t > 0)
        def _():
            for r in range(1, RING):
                wait_scatter((BLOCK - RING + r) % RING)

        plsc.subcore_barrier()

        pltpu.sync_copy(acc.at[pl.ds(base, rows_per_tile)],
                        part_hbm.at[cid, pl.ds(base, rows_per_tile)])
        if rem_rows:
            @pl.when(sid == NS - 1)
            def _():
                tail = NS * rows_per_tile
                pltpu.sync_copy(acc.at[pl.ds(tail, rem_rows)],
                                part_hbm.at[cid, pl.ds(tail, rem_rows)])

    return k(h, srcr, dstr, wr)


def kernel(x, edge_index, edge_weight, W, b):
    N, _ = x.shape
    D = W.shape[0]
    E = edge_weight.shape[0]

    h = _linear_tc(x, W, b)

    t_chunks = -(-E // CHUNK)

    def _blk_pt(chunks):
        pt = -(-chunks // NS)
        return -(-pt // BLOCK) * BLOCK

    n0 = _blk_pt(int(round(t_chunks * SPLIT0)))
    n1 = _blk_pt(max(t_chunks - NS * n0, 0))
    e_pad = NS * (n0 + n1) * CHUNK
    pad_n = e_pad - E
    pad_idx = jnp.asarray((np.arange(pad_n, dtype=np.int32) * 13) % N)
    dst = jnp.concatenate([edge_index[0], pad_idx])
    src = jnp.concatenate([edge_index[1], pad_idx])
    w = jnp.pad(edge_weight, (0, pad_n))
    srcr = src.reshape(-1, BLOCK, CHUNK)
    dstr = dst.reshape(-1, BLOCK, CHUNK)
    wr = w.reshape(-1, BLOCK, CHUNK)

    part = _aggregate_sc(h, srcr, dstr, wr, n0, n1, N, D)
    return _combine_tc(part)


# speedup vs baseline: 1.0785x; 1.0351x over previous
import functools

import jax
import jax.numpy as jnp
import numpy as np
from jax import lax
from jax.experimental import pallas as pl
from jax.experimental.pallas import tpu as pltpu
from jax.experimental.pallas import tpu_sc as plsc

NC = 2
NS = 16
NW = NC * NS
CHUNK = 112
BLOCK = 6
RING = 3
SPLIT0 = 0.51


def _linear_tc(x, W, b):
    N, D_in = x.shape
    D_out = W.shape[0]
    BLK = 1000
    grid = (N // BLK,)

    def body(x_ref, w_ref, b_ref, h_ref):
        acc = lax.dot_general(
            x_ref[...], w_ref[...],
            (((1,), (1,)), ((), ())),
            preferred_element_type=jnp.float32,
        )
        h_ref[...] = acc + b_ref[...][None, :]

    return pl.pallas_call(
        body,
        grid=grid,
        in_specs=[
            pl.BlockSpec((BLK, D_in), lambda i: (i, 0)),
            pl.BlockSpec((D_out, D_in), lambda i: (0, 0)),
            pl.BlockSpec((D_out,), lambda i: (0,)),
        ],
        out_specs=pl.BlockSpec((BLK, D_out), lambda i: (i, 0)),
        out_shape=jax.ShapeDtypeStruct((N, D_out), jnp.float32),
    )(x, W, b)


def _combine_tc(part):
    _, N, D = part.shape
    BLK = 1000
    grid = (N // BLK,)

    def body(p_ref, o_ref):
        o_ref[...] = p_ref[0] + p_ref[1]

    return pl.pallas_call(
        body,
        grid=grid,
        in_specs=[pl.BlockSpec((2, BLK, D), lambda i: (0, i, 0))],
        out_specs=pl.BlockSpec((BLK, D), lambda i: (i, 0)),
        out_shape=jax.ShapeDtypeStruct((N, D), jnp.float32),
    )(part)


def _aggregate_sc(h, ei, wr, n0, n1, N, D):
    rows_per_tile = (N // NS) // 8 * 8
    rem_rows = N - rows_per_tile * NS

    mesh = plsc.VectorSubcoreMesh(core_axis_name="c", subcore_axis_name="s",
                                  num_cores=NC, num_subcores=NS)

    @functools.partial(
        pl.kernel,
        out_type=jax.ShapeDtypeStruct((NC, N, D), jnp.float32),
        mesh=mesh,
        scratch_types=[
            pltpu.VMEM_SHARED((N, D), jnp.float32),
            pltpu.VMEM((2, BLOCK, CHUNK), jnp.int32),
            pltpu.VMEM((2, BLOCK, CHUNK), jnp.int32),
            pltpu.VMEM((2, BLOCK, CHUNK), jnp.float32),
            pltpu.VMEM((RING, CHUNK, D), jnp.float32),
            pltpu.SemaphoreType.DMA,
            pltpu.SemaphoreType.DMA,
            pltpu.SemaphoreType.DMA,
            pltpu.SemaphoreType.DMA,
            pltpu.SemaphoreType.DMA,
            pltpu.SemaphoreType.DMA,
            pltpu.SemaphoreType.DMA,
        ],
    )
    def k(h_hbm, ei_hbm, w_hbm, part_hbm,
          acc, src_v, dst_v, w_v, rows_v,
          sem_i, sem_g0, sem_g1, sem_g2, sem_s0, sem_s1, sem_s2):
        cid = lax.axis_index("c")
        sid = lax.axis_index("s")
        nb0 = n0 // BLOCK
        nb1 = n1 // BLOCK
        n_t = jnp.where(cid == 0, n0, n1)
        bstart = jnp.where(cid == 0, sid * nb0, NS * nb0 + sid * nb1)
        sem_g = (sem_g0, sem_g1, sem_g2)
        sem_s = (sem_s0, sem_s1, sem_s2)

        def issue_idxblk(k_):
            kb = k_ % 2
            pltpu.async_copy(ei_hbm.at[0, bstart + k_], src_v.at[kb], sem_i)
            pltpu.async_copy(ei_hbm.at[1, bstart + k_], dst_v.at[kb], sem_i)
            pltpu.async_copy(w_hbm.at[bstart + k_], w_v.at[kb], sem_i)

        def wait_idxblk():
            pltpu.make_async_copy(ei_hbm.at[0, 0], src_v.at[0], sem_i).wait()
            pltpu.make_async_copy(ei_hbm.at[1, 0], dst_v.at[0], sem_i).wait()
            pltpu.make_async_copy(w_hbm.at[0], w_v.at[0], sem_i).wait()

        def issue_gather(kb, j, b):
            pltpu.async_copy(h_hbm.at[src_v.at[kb, j]], rows_v.at[b],
                             sem_g[b])

        def wait_gather(b):
            pltpu.make_async_copy(h_hbm.at[pl.ds(0, CHUNK)], rows_v.at[b],
                                  sem_g[b]).wait()

        def issue_scatter(kb, j, b):
            pltpu.async_copy(rows_v.at[b], acc.at[dst_v.at[kb, j]],
                             sem_s[b], add=True)

        def wait_scatter(b):
            pltpu.make_async_copy(h_hbm.at[pl.ds(0, CHUNK)], rows_v.at[b],
                                  sem_s[b]).wait()

        @pl.when(n_t > 0)
        def _():
            issue_idxblk(0)

        def zfill(i, _):
            r = i // (D // 16)
            c = (i % (D // 16)) * 16
            rows_v[0, r, pl.ds(c, 16)] = jnp.zeros((16,), jnp.float32)
            return 0
        lax.fori_loop(0, CHUNK * (D // 16), zfill, 0)

        base = sid * rows_per_tile
        full = rows_per_tile // CHUNK
        rem = rows_per_tile - full * CHUNK
        zcopies = []
        for q in range(full):
            zcopies.append(pltpu.async_copy(
                rows_v.at[0], acc.at[pl.ds(base + q * CHUNK, CHUNK)],
                sem_g0))
        if rem:
            zcopies.append(pltpu.async_copy(
                rows_v.at[0, pl.ds(0, rem)],
                acc.at[pl.ds(base + full * CHUNK, rem)], sem_g0))
        if rem_rows:
            @pl.when(sid == NS - 1)
            def _():
                pltpu.sync_copy(rows_v.at[0, pl.ds(0, rem_rows)],
                                acc.at[pl.ds(NS * rows_per_tile, rem_rows)])
        for cp in zcopies:
            cp.wait()

        plsc.subcore_barrier()

        @pl.when(n_t > 0)
        def _():
            wait_idxblk()
            issue_gather(0, 0, 0)

        def scale_rows(kb, j, b):
            def group_body(g, _):
                wv16 = w_v[kb, j, pl.ds(g * 16, 16)]
                for t in range(16):
                    e = g * 16 + t
                    wgt = wv16[t]
                    for u in range(D // 16):
                        sl = pl.ds(u * 16, 16)
                        rows_v[b, e, sl] = rows_v[b, e, sl] * wgt
                return 0
            lax.fori_loop(0, CHUNK // 16, group_body, 0)

        def outer_body(k_, _):
            kb = k_ % 2
            for j in range(BLOCK):
                c = BLOCK * k_ + j
                b = j % RING
                nxt = (j + 1) % RING

                @pl.when(c + 1 >= RING)
                def _():
                    wait_scatter(nxt)

                if j == 0:
                    @pl.when(BLOCK * (k_ + 1) < n_t)
                    def _():
                        issue_idxblk(k_ + 1)

                if j == BLOCK - 1:
                    @pl.when(c + 1 < n_t)
                    def _():
                        wait_idxblk()

                @pl.when(c + 1 < n_t)
                def _():
                    if j == BLOCK - 1:
                        issue_gather(1 - kb, 0, nxt)
                    else:
                        issue_gather(kb, j + 1, nxt)

                wait_gather(b)
                scale_rows(kb, j, b)
                issue_scatter(kb, j, b)
            return 0
        lax.fori_loop(0, n_t // BLOCK, outer_body, 0)

        @pl.when(n_t > 0)
        def _():
            for r in range(1, RING):
                wait_scatter((BLOCK - RING + r) % RING)

        plsc.subcore_barrier()

        pltpu.sync_copy(acc.at[pl.ds(base, rows_per_tile)],
                        part_hbm.at[cid, pl.ds(base, rows_per_tile)])
        if rem_rows:
            @pl.when(sid == NS - 1)
            def _():
                tail = NS * rows_per_tile
                pltpu.sync_copy(acc.at[pl.ds(tail, rem_rows)],
                                part_hbm.at[cid, pl.ds(tail, rem_rows)])

    return k(h, ei, wr)


def kernel(x, edge_index, edge_weight, W, b):
    N, _ = x.shape
    D = W.shape[0]
    E = edge_weight.shape[0]

    h = _linear_tc(x, W, b)

    t_chunks = -(-E // CHUNK)

    def _blk_pt(chunks):
        pt = -(-chunks // NS)
        return -(-pt // BLOCK) * BLOCK

    n0 = _blk_pt(int(round(t_chunks * SPLIT0)))
    n1 = _blk_pt(max(t_chunks - NS * n0, 0))
    e_pad = NS * (n0 + n1) * CHUNK
    pad_n = e_pad - E
    pad_idx = jnp.asarray((np.arange(pad_n, dtype=np.int32) * 13) % N)
    ei = jnp.concatenate(
        [edge_index[1], pad_idx, edge_index[0], pad_idx]
    ).reshape(2, -1, BLOCK, CHUNK)
    wr = jnp.pad(edge_weight, (0, pad_n)).reshape(-1, BLOCK, CHUNK)

    part = _aggregate_sc(h, ei, wr, n0, n1, N, D)
    return _combine_tc(part)
